# Initial kernel scaffold; baseline (speedup 1.0000x reference)
import jax
import jax.numpy as jnp
from jax.experimental import pallas as pl


def kernel(all_emb, edge_index, graph_vals, W1, b1, W2, b2, rnd_idx, rnd_vals):
    raise NotImplementedError("write your pallas kernel here")


# speedup vs baseline: 22.5584x; 22.5584x over previous
import functools

import jax
import jax.numpy as jnp
import numpy as np
from jax import lax
from jax.experimental import pallas as pl
from jax.experimental.pallas import tpu as pltpu
from jax.experimental.pallas import tpu_sc as plsc

_LOGIT_99 = float(np.log(0.99) - np.log(0.01))

_E = 800000
_R = 100000
_TOTAL = _E + _R
_NW = 32
_EP = 802816
_EC = _EP // _NW
_EWIN = 512
_RP = 100352
_RC = _RP // _NW
_RWIN = 448
_TP = 917504
_PC = _TP // _NW
_PWIN = 2048
_TAILW = (_TP - _EP - _RP) // _NW
_NB = 2048
_SHIFTS = (0, 11, 22)
_OUTP = 901120
_OC = _OUTP // 16
_OWIN = 1024
_SPN = 917504
_DUMP = 901120
_MIN32 = -(2**31)

_OCC_BASE = 1


def _mesh():
    return plsc.VectorSubcoreMesh(core_axis_name="c", subcore_axis_name="s")


def _params():
    return pltpu.CompilerParams(use_tc_tiling_on_sc=False,
                                needs_layout_passes=False)


def _wid():
    return lax.axis_index("s") * 2 + lax.axis_index("c")


def _lanes():
    return lax.iota(jnp.int32, 16)


def _vsum(v):
    return jnp.sum(v, dtype=v.dtype)


def _ab_body(emb_ref, w1a_ref, w1b_ref, b1_ref, a_ref, b_ref):
    x = emb_ref[...]
    a_ref[...] = jnp.dot(x, w1a_ref[...], preferred_element_type=jnp.float32)
    b_ref[...] = (
        jnp.dot(x, w1b_ref[...], preferred_element_type=jnp.float32) + b1_ref[...]
    )


def _precompute_ab(all_emb, W1, b1):
    n, d = all_emb.shape
    blk = 2000
    return pl.pallas_call(
        _ab_body,
        grid=(n // blk,),
        in_specs=[
            pl.BlockSpec((blk, d), lambda i: (i, 0)),
            pl.BlockSpec((d, d), lambda i: (0, 0)),
            pl.BlockSpec((d, d), lambda i: (0, 0)),
            pl.BlockSpec((1, d), lambda i: (0, 0)),
        ],
        out_specs=[
            pl.BlockSpec((blk, d), lambda i: (i, 0)),
            pl.BlockSpec((blk, d), lambda i: (i, 0)),
        ],
        out_shape=[
            jax.ShapeDtypeStruct((n, d), jnp.float32),
            jax.ShapeDtypeStruct((n, d), jnp.float32),
        ],
    )(all_emb, W1[:d], W1[d:], b1.reshape(1, d))


def _noise_constants(e):
    bias = 0.0 + 0.0001
    u = jax.random.uniform(jax.random.key(123), (e, 1), dtype=jnp.float32)
    eps = (bias - (1.0 - bias)) * u + (1.0 - bias)
    gi = (jnp.log(eps) - jnp.log(1.0 - eps)).squeeze(-1)
    u2 = jax.random.uniform(
        jax.random.key(456), (e,), minval=1e-6, maxval=1.0 - 1e-6, dtype=jnp.float32
    )
    l2 = jnp.log(u2) - jnp.log(1.0 - u2)
    return gi, l2


def _hist_kernel(shift, keys_h, hist_h, win_k, hist_v):
    w = _wid()

    def zh(i, _):
        hist_v[pl.ds(i * 16, 16)] = jnp.zeros((16,), jnp.int32)
        return 0
    lax.fori_loop(0, _NB // 16, zh, 0)

    cbase0 = w * _PC

    def window(win, _):
        base = cbase0 + win * _PWIN
        pltpu.sync_copy(keys_h.at[pl.ds(base, _PWIN)], win_k)

        def vreg(i, _):
            k = win_k[pl.ds(i * 16, 16)]
            dig = lax.shift_right_logical(k, jnp.int32(shift)) & jnp.int32(_NB - 1)
            occ, last = plsc.scan_count(dig)
            cnt = occ + jnp.int32(1 - _OCC_BASE)
            plsc.addupdate_scatter(hist_v, [dig], cnt, mask=last)
            return 0
        lax.fori_loop(0, _PWIN // 16, vreg, 0)
        return 0

    lax.fori_loop(0, _PC // _PWIN, window, 0)
    pltpu.sync_copy(hist_v, hist_h.at[w])


def _run_hist(shift, keys):
    kern = pl.kernel(
        functools.partial(_hist_kernel, shift),
        compiler_params=_params(),
        out_type=jax.ShapeDtypeStruct((_NW, _NB), jnp.int32),
        mesh=_mesh(),
        scratch_types=[
            pltpu.VMEM((_PWIN,), jnp.int32),
            pltpu.VMEM((_NB,), jnp.int32),
        ],
    )
    return kern(keys)


def _edges_kernel(a_tab, b_tab, src_h, dst_h, gi_h, l2_h, w2_h, b2_h,
                  ru_h, ri_h, rv_h,
                  keys_h, vals_h,
                  idx_s, idx_d, a_rows, b_rows, gi_v, l2_v, w2_v, b2_v,
                  keys_v, vals_v, sem):
    w = _wid()
    lanes = _lanes()

    pltpu.sync_copy(w2_h, w2_v)
    pltpu.sync_copy(b2_h, b2_v)
    b2s = b2_v[pl.ds(0, 16)]

    ebase0 = w * _EC

    def edge_window(win, _):
        base = ebase0 + win * _EWIN
        pltpu.sync_copy(src_h.at[pl.ds(base, _EWIN)], idx_s)
        pltpu.sync_copy(dst_h.at[pl.ds(base, _EWIN)], idx_d)
        pltpu.sync_copy(gi_h.at[pl.ds(base, _EWIN)], gi_v)
        pltpu.sync_copy(l2_h.at[pl.ds(base, _EWIN)], l2_v)
        cps = []
        for j in range(_EWIN // 128):
            cps.append(pltpu.async_copy(
                a_tab.at[idx_s.at[pl.ds(j * 128, 128)]],
                a_rows.at[pl.ds(j * 128, 128)], sem))
            cps.append(pltpu.async_copy(
                b_tab.at[idx_d.at[pl.ds(j * 128, 128)]],
                b_rows.at[pl.ds(j * 128, 128)], sem))
        for cp in cps:
            cp.wait()

        def group(g, _):
            e0 = g * 16
            acc = jnp.zeros((16,), jnp.float32)
            for eu in range(16):
                e = e0 + eu
                s = jnp.zeros((16,), jnp.float32)
                for f in range(4):
                    av = a_rows[e, pl.ds(f * 16, 16)]
                    bv = b_rows[e, pl.ds(f * 16, 16)]
                    h = jnp.maximum(av + bv, 0.0)
                    s = s + h * w2_v[pl.ds(f * 16, 16)]
                tot = _vsum(s)
                acc = acc + jnp.where(lanes == jnp.int32(eu),
                                      jnp.full((16,), tot, jnp.float32),
                                      jnp.zeros((16,), jnp.float32))
            giv = gi_v[pl.ds(e0, 16)]
            l2v = l2_v[pl.ds(e0, 16)]
            x = giv + acc + b2s
            lp = jnp.clip(x, -_LOGIT_99, _LOGIT_99)
            z = (lp + l2v) * jnp.float32(1.0 / 0.9)
            val = 1.0 / (1.0 + jnp.exp(-z))
            sv = idx_s[pl.ds(e0, 16)]
            dv = idx_d[pl.ds(e0, 16)]
            key = lax.shift_left(sv, jnp.int32(16)) | dv
            gidx = jnp.full((16,), base, jnp.int32) + jnp.int32(e0) + lanes
            real = gidx < jnp.int32(_E)
            key = jnp.where(real, key, jnp.full((16,), -1, jnp.int32))
            val = jnp.where(real, val, jnp.zeros((16,), jnp.float32))
            keys_v[pl.ds(e0, 16)] = key
            vals_v[pl.ds(e0, 16)] = val
            return 0

        lax.fori_loop(0, _EWIN // 16, group, 0)
        pltpu.sync_copy(keys_v, keys_h.at[pl.ds(base, _EWIN)])
        pltpu.sync_copy(vals_v, vals_h.at[pl.ds(base, _EWIN)])
        return 0

    lax.fori_loop(0, _EC // _EWIN, edge_window, 0)

    rbase0 = w * _RC

    def rnd_window(win, _):
        base = rbase0 + win * _RWIN
        pltpu.sync_copy(ru_h.at[pl.ds(base, _RWIN)], idx_s.at[pl.ds(0, _RWIN)])
        pltpu.sync_copy(ri_h.at[pl.ds(base, _RWIN)], idx_d.at[pl.ds(0, _RWIN)])
        pltpu.sync_copy(rv_h.at[pl.ds(base, _RWIN)], gi_v.at[pl.ds(0, _RWIN)])

        def group(g, _):
            e0 = g * 16
            uv = idx_s[pl.ds(e0, 16)]
            iv = idx_d[pl.ds(e0, 16)]
            key = lax.shift_left(uv, jnp.int32(16)) | iv
            val = gi_v[pl.ds(e0, 16)]
            gidx = jnp.full((16,), base, jnp.int32) + jnp.int32(e0) + lanes
            real = gidx < jnp.int32(_R)
            key = jnp.where(real, key, jnp.full((16,), -1, jnp.int32))
            val = jnp.where(real, val, jnp.zeros((16,), jnp.float32))
            keys_v[pl.ds(e0, 16)] = key
            vals_v[pl.ds(e0, 16)] = val
            return 0

        lax.fori_loop(0, _RWIN // 16, group, 0)
        pltpu.sync_copy(keys_v.at[pl.ds(0, _RWIN)],
                        keys_h.at[pl.ds(_EP + base, _RWIN)])
        pltpu.sync_copy(vals_v.at[pl.ds(0, _RWIN)],
                        vals_h.at[pl.ds(_EP + base, _RWIN)])
        return 0

    lax.fori_loop(0, _RC // _RWIN, rnd_window, 0)

    def tail_group(g, _):
        e0 = g * 16
        keys_v[pl.ds(e0, 16)] = jnp.full((16,), -1, jnp.int32)
        vals_v[pl.ds(e0, 16)] = jnp.zeros((16,), jnp.float32)
        return 0
    lax.fori_loop(0, _TAILW // 16, tail_group, 0)
    pltpu.sync_copy(keys_v.at[pl.ds(0, _TAILW)],
                    keys_h.at[pl.ds(_EP + _RP + w * _TAILW, _TAILW)])
    pltpu.sync_copy(vals_v.at[pl.ds(0, _TAILW)],
                    vals_h.at[pl.ds(_EP + _RP + w * _TAILW, _TAILW)])


def _run_edges(a_tab, b_tab, src_p, dst_p, gi_p, l2_p, w2f, b2v, ru, ri, rv):
    kern = pl.kernel(
        _edges_kernel,
        compiler_params=_params(),
        out_type=(
            jax.ShapeDtypeStruct((_TP,), jnp.int32),
            jax.ShapeDtypeStruct((_TP,), jnp.float32),
        ),
        mesh=_mesh(),
        scratch_types=[
            pltpu.VMEM((_EWIN,), jnp.int32),
            pltpu.VMEM((_EWIN,), jnp.int32),
            pltpu.VMEM((_EWIN, 64), jnp.float32),
            pltpu.VMEM((_EWIN, 64), jnp.float32),
            pltpu.VMEM((_EWIN,), jnp.float32),
            pltpu.VMEM((_EWIN,), jnp.float32),
            pltpu.VMEM((64,), jnp.float32),
            pltpu.VMEM((16,), jnp.float32),
            pltpu.VMEM((_EWIN,), jnp.int32),
            pltpu.VMEM((_EWIN,), jnp.float32),
            pltpu.SemaphoreType.DMA,
        ],
    )
    return kern(a_tab, b_tab, src_p, dst_p, gi_p, l2_p, w2f, b2v, ru, ri, rv)


def _permute_kernel(shift, keys_in, vals_in, hist_h, keys_out, vals_out,
                    hist_v, t_v, a_v, counters, win_k, win_x, dest_v, sem):
    w = _wid()
    lanes = _lanes()

    pltpu.sync_copy(hist_h, hist_v)

    def colsum(cch, _):
        acc_all = jnp.zeros((16,), jnp.int32)
        acc_my = jnp.zeros((16,), jnp.int32)
        for wp in range(_NW):
            v = hist_v[wp, pl.ds(cch * 16, 16)]
            acc_all = acc_all + v
            m = (jnp.int32(wp) < w).astype(jnp.int32)
            acc_my = acc_my + v * m
        t_v[pl.ds(cch * 16, 16)] = acc_all
        a_v[pl.ds(cch * 16, 16)] = acc_my
        return 0
    lax.fori_loop(0, _NB // 16, colsum, 0)

    def prefix(cch, carry):
        t = t_v[pl.ds(cch * 16, 16)]
        inc = plsc.cumsum(t)
        excl = inc - t + jnp.full((16,), carry, jnp.int32)
        counters[pl.ds(cch * 16, 16)] = excl + a_v[pl.ds(cch * 16, 16)]
        return carry + _vsum(t)
    lax.fori_loop(0, _NB // 16, prefix, jnp.int32(0))

    cbase0 = w * _PC

    def window(win, _):
        base = cbase0 + win * _PWIN
        pltpu.sync_copy(keys_in.at[pl.ds(base, _PWIN)], win_k)
        pltpu.sync_copy(vals_in.at[pl.ds(base, _PWIN)], win_x)
        for j in range(_PWIN // 128):
            for jj in range(8):
                i = j * 8 + jj
                k = win_k[pl.ds(i * 16, 16)]
                dig = lax.shift_right_logical(k, jnp.int32(shift)) & jnp.int32(_NB - 1)
                occ, last = plsc.scan_count(dig)
                b = plsc.load_gather(counters, [dig])
                dest = b + occ + jnp.int32(-_OCC_BASE)
                plsc.store_scatter(counters, [dig],
                                   b + occ + jnp.int32(1 - _OCC_BASE), mask=last)
                dest_v[j, pl.ds(jj * 16, 16)] = dest
        cps = []
        for j in range(_PWIN // 128):
            cps.append(pltpu.async_copy(
                win_k.at[pl.ds(j * 128, 128)],
                keys_out.at[dest_v.at[j]], sem))
            cps.append(pltpu.async_copy(
                win_x.at[pl.ds(j * 128, 128)],
                vals_out.at[dest_v.at[j]], sem))
        for cp in cps:
            cp.wait()
        return 0

    lax.fori_loop(0, _PC // _PWIN, window, 0)


def _run_permute(shift, keys_in, vals_in, hist_p):
    kern = pl.kernel(
        functools.partial(_permute_kernel, shift),
        compiler_params=_params(),
        out_type=(
            jax.ShapeDtypeStruct((_TP,), jnp.int32),
            jax.ShapeDtypeStruct((_TP,), jnp.float32),
        ),
        mesh=_mesh(),
        scratch_types=[
            pltpu.VMEM((_NW, _NB), jnp.int32),
            pltpu.VMEM((_NB,), jnp.int32),
            pltpu.VMEM((_NB,), jnp.int32),
            pltpu.VMEM((_NB,), jnp.int32),
            pltpu.VMEM((_PWIN,), jnp.int32),
            pltpu.VMEM((_PWIN,), jnp.float32),
            pltpu.VMEM((_PWIN // 128, 128), jnp.int32),
            pltpu.SemaphoreType.DMA,
        ],
    )
    return kern(keys_in, vals_in, hist_p)


def _coalesce_kernel(n_total, keys_h, xvals_h, vals_h, lo_h, hi_h,
                     win_k, win_x, dest_v, cnt_v, vbuf, lbuf, zero_v, zero_i,
                     vacc_sp, lacc_sp, cnt_sp, sem):
    c = lax.axis_index("c")
    s = lax.axis_index("s")
    lanes = _lanes()
    zeros16 = jnp.zeros((16,), jnp.int32)

    @pl.when(c == 0)
    def _():
        def zv(i, _):
            zero_v[pl.ds(i * 16, 16)] = jnp.zeros((16,), jnp.float32)
            zero_i[pl.ds(i * 16, 16)] = jnp.zeros((16,), jnp.int32)
            return 0
        lax.fori_loop(0, _OWIN // 16, zv, 0)

        def zwin(i, _):
            off = s * (_SPN // 16) + i * _OWIN
            pltpu.sync_copy(zero_v, vacc_sp.at[pl.ds(off, _OWIN)])
            pltpu.sync_copy(zero_i, lacc_sp.at[pl.ds(off, _OWIN)])
            return 0
        lax.fori_loop(0, _SPN // 16 // _OWIN, zwin, 0)

        cbase0 = s * (_TP // 16)

        @pl.when(s > 0)
        def _():
            pltpu.sync_copy(keys_h.at[pl.ds(cbase0 - 8, 8)],
                            win_k.at[pl.ds(0, 8)])
        prevv = win_k[pl.ds(0, 16)]
        prev0 = jnp.where(
            (jnp.full((16,), 1, jnp.int32) * (s > 0).astype(jnp.int32)) > 0,
            jnp.full((16,), _vsum(jnp.where(lanes == 7, prevv, zeros16)),
                     jnp.int32),
            jnp.full((16,), -2, jnp.int32))

        shiftperm = jnp.maximum(lanes - 1, 0)
        gdn = lax.GatherDimensionNumbers(
            offset_dims=(), collapsed_slice_dims=(0,), start_index_map=(0,))

        def boundaries(k, prev_carry, base_i):
            kshift = lax.gather(k, shiftperm.reshape(16, 1), gdn, (1,),
                                mode=lax.GatherScatterMode.PROMISE_IN_BOUNDS)
            prev = jnp.where(lanes == 0, jnp.full((16,), prev_carry, jnp.int32),
                             kshift)
            gpos = jnp.full((16,), base_i, jnp.int32) + lanes
            isn = ((k != prev) & (gpos > 0) & (gpos < jnp.int32(_TOTAL)))
            new_carry = _vsum(jnp.where(lanes == 15, k, zeros16))
            return isn.astype(jnp.int32), new_carry

        p0 = _vsum(jnp.where(lanes == 0, prev0, zeros16))

        def awin(win, carry):
            nb, pk = carry
            base = cbase0 + win * _PWIN
            pltpu.sync_copy(keys_h.at[pl.ds(base, _PWIN)], win_k)

            def avreg(i, carry2):
                nb2, pk2 = carry2
                k = win_k[pl.ds(i * 16, 16)]
                isn, pk3 = boundaries(k, pk2, base + i * 16)
                return nb2 + _vsum(isn), pk3
            return lax.fori_loop(0, _PWIN // 16, avreg, (nb, pk))

        nbound, _pk = lax.fori_loop(0, _TP // 16 // _PWIN, awin,
                                    (jnp.int32(0), p0))

        cnt_v[0, pl.ds(0, 16)] = jnp.full((16,), nbound, jnp.int32)
        pltpu.sync_copy(cnt_v.at[0], cnt_sp.at[s])
        plsc.subcore_barrier()
        pltpu.sync_copy(cnt_sp, cnt_v)

        offs = jnp.int32(0)
        nseg = jnp.int32(0)
        for ws in range(16):
            v = cnt_v[ws, pl.ds(0, 16)]
            cw = _vsum(jnp.where(lanes == 0, v, zeros16))
            offs = offs + cw * (jnp.int32(ws) < s).astype(jnp.int32)
            nseg = nseg + cw
        nseg = nseg + 1

        def bwin(win, carry):
            seg0, pk = carry
            base = cbase0 + win * _PWIN
            pltpu.sync_copy(keys_h.at[pl.ds(base, _PWIN)], win_k)
            pltpu.sync_copy(xvals_h.at[pl.ds(base, _PWIN)], win_x)

            def bvreg(i, carry2):
                seg1, pk2 = carry2
                k = win_k[pl.ds(i * 16, 16)]
                v = win_x[pl.ds(i * 16, 16)]
                isn, pk3 = boundaries(k, pk2, base + i * 16)
                segv = jnp.full((16,), seg1, jnp.int32) + plsc.cumsum(isn)
                gpos = jnp.full((16,), base + i * 16, jnp.int32) + lanes
                valid = gpos < jnp.int32(_TOTAL)
                sv = lax.shift_right_logical(k, jnp.int32(16))
                dv = k & jnp.int32(0xFFFF)
                lo = sv * jnp.int32(n_total) + dv
                dump = jnp.full((16,), _DUMP, jnp.int32) + (gpos & jnp.int32(255))
                dest = jnp.where(valid, segv, dump)
                vbuf[pl.ds(i * 16, 16)] = jnp.where(
                    valid, v, jnp.zeros((16,), jnp.float32))
                lbuf[pl.ds(i * 16, 16)] = lo
                jrow = lax.shift_right_logical(i, 3)
                dest_v[jrow, pl.ds((i & 7) * 16, 16)] = dest
                return seg1 + _vsum(isn), pk3

            seg2, pk2 = lax.fori_loop(0, _PWIN // 16, bvreg, (seg0, pk))
            cps = []
            for j in range(_PWIN // 128):
                cps.append(pltpu.async_copy(
                    vbuf.at[pl.ds(j * 128, 128)],
                    vacc_sp.at[dest_v.at[j]], sem, add=True))
                cps.append(pltpu.async_copy(
                    lbuf.at[pl.ds(j * 128, 128)],
                    lacc_sp.at[dest_v.at[j]], sem))
            for cp in cps:
                cp.wait()
            return seg2, pk2

        lax.fori_loop(0, _TP // 16 // _PWIN, bwin, (offs, p0))
        plsc.subcore_barrier()

        def cwin(i, _):
            off = s * _OC + i * _OWIN
            pltpu.sync_copy(vacc_sp.at[pl.ds(off, _OWIN)],
                            vbuf.at[pl.ds(0, _OWIN)])
            pltpu.sync_copy(vbuf.at[pl.ds(0, _OWIN)],
                            vals_h.at[pl.ds(off, _OWIN)])
            pltpu.sync_copy(lacc_sp.at[pl.ds(off, _OWIN)],
                            lbuf.at[pl.ds(0, _OWIN)])
            pltpu.sync_copy(lbuf.at[pl.ds(0, _OWIN)],
                            lo_h.at[pl.ds(off, _OWIN)])

            def hveq(q, _):
                gpos = jnp.full((16,), off + q * 16, jnp.int32) + lanes
                hi = jnp.where(gpos < nseg, zeros16,
                               jnp.full((16,), _MIN32, jnp.int32))
                lbuf[pl.ds(q * 16, 16)] = hi
                return 0
            lax.fori_loop(0, _OWIN // 16, hveq, 0)
            pltpu.sync_copy(lbuf.at[pl.ds(0, _OWIN)], hi_h.at[pl.ds(off, _OWIN)])
            return 0
        lax.fori_loop(0, _OC // _OWIN, cwin, 0)


def _run_coalesce(n_total, keys, xvals):
    kern = pl.kernel(
        functools.partial(_coalesce_kernel, n_total),
        compiler_params=_params(),
        out_type=(
            jax.ShapeDtypeStruct((_OUTP,), jnp.float32),
            jax.ShapeDtypeStruct((_OUTP,), jnp.int32),
            jax.ShapeDtypeStruct((_OUTP,), jnp.int32),
        ),
        mesh=_mesh(),
        scratch_types=[
            pltpu.VMEM((_PWIN,), jnp.int32),
            pltpu.VMEM((_PWIN,), jnp.float32),
            pltpu.VMEM((16, 128), jnp.int32),
            pltpu.VMEM((16, 16), jnp.int32),
            pltpu.VMEM((_PWIN,), jnp.float32),
            pltpu.VMEM((_PWIN,), jnp.int32),
            pltpu.VMEM((_OWIN,), jnp.float32),
            pltpu.VMEM((_OWIN,), jnp.int32),
            pltpu.VMEM_SHARED((_SPN,), jnp.float32),
            pltpu.VMEM_SHARED((_SPN,), jnp.int32),
            pltpu.VMEM_SHARED((16, 16), jnp.int32),
            pltpu.SemaphoreType.DMA,
        ],
    )
    return kern(keys, xvals)


def kernel(all_emb, edge_index, graph_vals, W1, b1, W2, b2, rnd_idx, rnd_vals):
    n, d = all_emb.shape
    e = edge_index.shape[1]
    r = rnd_idx.shape[1]
    total = e + r

    with jax.enable_x64(False):
        gi, l2 = _noise_constants(e)
        a_tab, b_tab = _precompute_ab(all_emb, W1, b1)

        src_p = jnp.pad(edge_index[0].astype(jnp.int32), (0, _EP - e))
        dst_p = jnp.pad(edge_index[1].astype(jnp.int32), (0, _EP - e))
        gi_p = jnp.pad(gi, (0, _EP - e))
        l2_p = jnp.pad(l2, (0, _EP - e))
        ru = jnp.pad(rnd_idx[0].astype(jnp.int32), (0, _RP - r))
        ri = jnp.pad(rnd_idx[1].astype(jnp.int32), (0, _RP - r))
        rv = jnp.pad(rnd_vals.astype(jnp.float32), (0, _RP - r))
        w2f = W2[:, 0].astype(jnp.float32)
        b2v = jnp.full((16,), b2[0], jnp.float32)

        keys, xvals = _run_edges(a_tab, b_tab, src_p, dst_p, gi_p, l2_p,
                                 w2f, b2v, ru, ri, rv)
        for sh in _SHIFTS:
            hist = _run_hist(sh, keys)
            keys, xvals = _run_permute(sh, keys, xvals, hist)

        vals_p, lo_p, hi_p = _run_coalesce(n, keys, xvals)

    out_vals = vals_p[:total]
    lin = (hi_p[:total].astype(jnp.int64) << 32) | (
        lo_p[:total].astype(jnp.uint32).astype(jnp.int64))
    return out_vals, lin


# speedup vs baseline: 94.3680x; 4.1833x over previous
import functools

import jax
import jax.numpy as jnp
import numpy as np
from jax import lax
from jax.experimental import pallas as pl
from jax.experimental.pallas import tpu as pltpu
from jax.experimental.pallas import tpu_sc as plsc

_LOGIT_99 = float(np.log(0.99) - np.log(0.01))

_E = 800000
_R = 100000
_TOTAL = _E + _R
_NW = 32
_EP = 802816
_EC = _EP // _NW
_EWIN = 512
_RP = 100352
_RC = _RP // _NW
_RWIN = 448
_TP = 917504
_PC = _TP // _NW
_PWIN = 2048
_TAILW = (_TP - _EP - _RP) // _NW
_NB = 2048
_SHIFTS = (0, 11, 22)
_OUTP = 901120
_OC = _OUTP // 16
_OWIN = 1024
_SPN = 917504
_DUMP = 901120
_MIN32 = -(2**31)

_OCC_BASE = 1


def _mesh():
    return plsc.VectorSubcoreMesh(core_axis_name="c", subcore_axis_name="s")


def _params():
    return pltpu.CompilerParams(use_tc_tiling_on_sc=False,
                                needs_layout_passes=False)


def _wid():
    return lax.axis_index("s") * 2 + lax.axis_index("c")


def _lanes():
    return lax.iota(jnp.int32, 16)


def _vsum(v):
    return jnp.sum(v, dtype=v.dtype)


def _ab_body(emb_ref, w1a_ref, w1b_ref, b1_ref, a_ref, b_ref):
    x = emb_ref[...]
    a_ref[...] = jnp.dot(x, w1a_ref[...], preferred_element_type=jnp.float32)
    b_ref[...] = (
        jnp.dot(x, w1b_ref[...], preferred_element_type=jnp.float32) + b1_ref[...]
    )


def _precompute_ab(all_emb, W1, b1):
    n, d = all_emb.shape
    blk = 2000
    return pl.pallas_call(
        _ab_body,
        grid=(n // blk,),
        in_specs=[
            pl.BlockSpec((blk, d), lambda i: (i, 0)),
            pl.BlockSpec((d, d), lambda i: (0, 0)),
            pl.BlockSpec((d, d), lambda i: (0, 0)),
            pl.BlockSpec((1, d), lambda i: (0, 0)),
        ],
        out_specs=[
            pl.BlockSpec((blk, d), lambda i: (i, 0)),
            pl.BlockSpec((blk, d), lambda i: (i, 0)),
        ],
        out_shape=[
            jax.ShapeDtypeStruct((n, d), jnp.float32),
            jax.ShapeDtypeStruct((n, d), jnp.float32),
        ],
    )(all_emb, W1[:d], W1[d:], b1.reshape(1, d))


def _noise_constants(e):
    bias = 0.0 + 0.0001
    u = jax.random.uniform(jax.random.key(123), (e, 1), dtype=jnp.float32)
    eps = (bias - (1.0 - bias)) * u + (1.0 - bias)
    gi = (jnp.log(eps) - jnp.log(1.0 - eps)).squeeze(-1)
    u2 = jax.random.uniform(
        jax.random.key(456), (e,), minval=1e-6, maxval=1.0 - 1e-6, dtype=jnp.float32
    )
    l2 = jnp.log(u2) - jnp.log(1.0 - u2)
    return gi, l2


def _hist_kernel(shift, keys_h, hist_h, win_k, hist_v):
    w = _wid()

    def zh(i, _):
        hist_v[pl.ds(i * 16, 16)] = jnp.zeros((16,), jnp.int32)
        return 0
    lax.fori_loop(0, _NB // 16, zh, 0)

    cbase0 = w * _PC

    def window(win, _):
        base = cbase0 + win * _PWIN
        pltpu.sync_copy(keys_h.at[pl.ds(base, _PWIN)], win_k)

        def vreg(i, _):
            k = win_k[pl.ds(i * 16, 16)]
            dig = lax.shift_right_logical(k, jnp.int32(shift)) & jnp.int32(_NB - 1)
            occ, last = plsc.scan_count(dig)
            cnt = occ + jnp.int32(1 - _OCC_BASE)
            plsc.addupdate_scatter(hist_v, [dig], cnt, mask=last)
            return 0
        lax.fori_loop(0, _PWIN // 16, vreg, 0)
        return 0

    lax.fori_loop(0, _PC // _PWIN, window, 0)
    pltpu.sync_copy(hist_v, hist_h.at[w])


def _run_hist(shift, keys):
    kern = pl.kernel(
        functools.partial(_hist_kernel, shift),
        compiler_params=_params(),
        out_type=jax.ShapeDtypeStruct((_NW, _NB), jnp.int32),
        mesh=_mesh(),
        scratch_types=[
            pltpu.VMEM((_PWIN,), jnp.int32),
            pltpu.VMEM((_NB,), jnp.int32),
        ],
    )
    return kern(keys)


def _edges_kernel(a_tab, b_tab, src_h, dst_h, gi_h, l2_h, w2_h, b2_h,
                  ru_h, ri_h, rv_h,
                  keys_h, vals_h,
                  idx_s, idx_d, a_rows, b_rows, gi_v, l2_v, w2_v, b2_v,
                  keys_v, vals_v, sem):
    w = _wid()
    lanes = _lanes()

    pltpu.sync_copy(w2_h, w2_v)
    pltpu.sync_copy(b2_h, b2_v)
    b2s = b2_v[pl.ds(0, 16)]

    ebase0 = w * _EC

    def edge_window(win, _):
        base = ebase0 + win * _EWIN
        pltpu.sync_copy(src_h.at[pl.ds(base, _EWIN)], idx_s)
        pltpu.sync_copy(dst_h.at[pl.ds(base, _EWIN)], idx_d)
        pltpu.sync_copy(gi_h.at[pl.ds(base, _EWIN)], gi_v)
        pltpu.sync_copy(l2_h.at[pl.ds(base, _EWIN)], l2_v)
        cps = []
        for j in range(_EWIN // 128):
            cps.append(pltpu.async_copy(
                a_tab.at[idx_s.at[pl.ds(j * 128, 128)]],
                a_rows.at[pl.ds(j * 128, 128)], sem))
            cps.append(pltpu.async_copy(
                b_tab.at[idx_d.at[pl.ds(j * 128, 128)]],
                b_rows.at[pl.ds(j * 128, 128)], sem))
        for cp in cps:
            cp.wait()

        def group(g, _):
            e0 = g * 16
            acc = jnp.zeros((16,), jnp.float32)
            for eu in range(16):
                e = e0 + eu
                s = jnp.zeros((16,), jnp.float32)
                for f in range(4):
                    av = a_rows[e, pl.ds(f * 16, 16)]
                    bv = b_rows[e, pl.ds(f * 16, 16)]
                    h = jnp.maximum(av + bv, 0.0)
                    s = s + h * w2_v[pl.ds(f * 16, 16)]
                tot = _vsum(s)
                acc = acc + jnp.where(lanes == jnp.int32(eu),
                                      jnp.full((16,), tot, jnp.float32),
                                      jnp.zeros((16,), jnp.float32))
            giv = gi_v[pl.ds(e0, 16)]
            l2v = l2_v[pl.ds(e0, 16)]
            x = giv + acc + b2s
            lp = jnp.clip(x, -_LOGIT_99, _LOGIT_99)
            z = (lp + l2v) * jnp.float32(1.0 / 0.9)
            val = 1.0 / (1.0 + jnp.exp(-z))
            sv = idx_s[pl.ds(e0, 16)]
            dv = idx_d[pl.ds(e0, 16)]
            key = lax.shift_left(sv, jnp.int32(16)) | dv
            gidx = jnp.full((16,), base, jnp.int32) + jnp.int32(e0) + lanes
            real = gidx < jnp.int32(_E)
            key = jnp.where(real, key, jnp.full((16,), -1, jnp.int32))
            val = jnp.where(real, val, jnp.zeros((16,), jnp.float32))
            keys_v[pl.ds(e0, 16)] = key
            vals_v[pl.ds(e0, 16)] = val
            return 0

        lax.fori_loop(0, _EWIN // 16, group, 0)
        pltpu.sync_copy(keys_v, keys_h.at[pl.ds(base, _EWIN)])
        pltpu.sync_copy(vals_v, vals_h.at[pl.ds(base, _EWIN)])
        return 0

    lax.fori_loop(0, _EC // _EWIN, edge_window, 0)

    rbase0 = w * _RC

    def rnd_window(win, _):
        base = rbase0 + win * _RWIN
        pltpu.sync_copy(ru_h.at[pl.ds(base, _RWIN)], idx_s.at[pl.ds(0, _RWIN)])
        pltpu.sync_copy(ri_h.at[pl.ds(base, _RWIN)], idx_d.at[pl.ds(0, _RWIN)])
        pltpu.sync_copy(rv_h.at[pl.ds(base, _RWIN)], gi_v.at[pl.ds(0, _RWIN)])

        def group(g, _):
            e0 = g * 16
            uv = idx_s[pl.ds(e0, 16)]
            iv = idx_d[pl.ds(e0, 16)]
            key = lax.shift_left(uv, jnp.int32(16)) | iv
            val = gi_v[pl.ds(e0, 16)]
            gidx = jnp.full((16,), base, jnp.int32) + jnp.int32(e0) + lanes
            real = gidx < jnp.int32(_R)
            key = jnp.where(real, key, jnp.full((16,), -1, jnp.int32))
            val = jnp.where(real, val, jnp.zeros((16,), jnp.float32))
            keys_v[pl.ds(e0, 16)] = key
            vals_v[pl.ds(e0, 16)] = val
            return 0

        lax.fori_loop(0, _RWIN // 16, group, 0)
        pltpu.sync_copy(keys_v.at[pl.ds(0, _RWIN)],
                        keys_h.at[pl.ds(_EP + base, _RWIN)])
        pltpu.sync_copy(vals_v.at[pl.ds(0, _RWIN)],
                        vals_h.at[pl.ds(_EP + base, _RWIN)])
        return 0

    lax.fori_loop(0, _RC // _RWIN, rnd_window, 0)

    def tail_group(g, _):
        e0 = g * 16
        keys_v[pl.ds(e0, 16)] = jnp.full((16,), -1, jnp.int32)
        vals_v[pl.ds(e0, 16)] = jnp.zeros((16,), jnp.float32)
        return 0
    lax.fori_loop(0, _TAILW // 16, tail_group, 0)
    pltpu.sync_copy(keys_v.at[pl.ds(0, _TAILW)],
                    keys_h.at[pl.ds(_EP + _RP + w * _TAILW, _TAILW)])
    pltpu.sync_copy(vals_v.at[pl.ds(0, _TAILW)],
                    vals_h.at[pl.ds(_EP + _RP + w * _TAILW, _TAILW)])


def _run_edges(a_tab, b_tab, src_p, dst_p, gi_p, l2_p, w2f, b2v, ru, ri, rv):
    kern = pl.kernel(
        _edges_kernel,
        compiler_params=_params(),
        out_type=(
            jax.ShapeDtypeStruct((_TP,), jnp.int32),
            jax.ShapeDtypeStruct((_TP,), jnp.float32),
        ),
        mesh=_mesh(),
        scratch_types=[
            pltpu.VMEM((_EWIN,), jnp.int32),
            pltpu.VMEM((_EWIN,), jnp.int32),
            pltpu.VMEM((_EWIN, 64), jnp.float32),
            pltpu.VMEM((_EWIN, 64), jnp.float32),
            pltpu.VMEM((_EWIN,), jnp.float32),
            pltpu.VMEM((_EWIN,), jnp.float32),
            pltpu.VMEM((64,), jnp.float32),
            pltpu.VMEM((16,), jnp.float32),
            pltpu.VMEM((_EWIN,), jnp.int32),
            pltpu.VMEM((_EWIN,), jnp.float32),
            pltpu.SemaphoreType.DMA,
        ],
    )
    return kern(a_tab, b_tab, src_p, dst_p, gi_p, l2_p, w2f, b2v, ru, ri, rv)


_HALF = _TP // 2
_EXPW = _HALF // 16


def _permute_kernel(shift, keys_in, vals_in, hist_h, keys_out, vals_out,
                    hist_v, t_v, a_v, counters, win_k, win_x, dest_v,
                    keys_sp, vals_sp, sem):
    c = lax.axis_index("c")
    s = lax.axis_index("s")
    lanes = _lanes()

    def colblock(bb, _):
        pltpu.sync_copy(hist_h.at[:, pl.ds(bb * 256, 256)], hist_v)

        def colsum(c2, _):
            acc_all = jnp.zeros((16,), jnp.int32)
            acc_my = jnp.zeros((16,), jnp.int32)
            for wp in range(_NW):
                v = hist_v[wp, pl.ds(c2 * 16, 16)]
                acc_all = acc_all + v
                m = (jnp.int32(wp) < s * 2).astype(jnp.int32)
                acc_my = acc_my + v * m
            t_v[pl.ds(bb * 256 + c2 * 16, 16)] = acc_all
            a_v[pl.ds(bb * 256 + c2 * 16, 16)] = acc_my
            return 0
        lax.fori_loop(0, 16, colsum, 0)
        return 0
    lax.fori_loop(0, _NB // 256, colblock, 0)

    def prefix(cch, carry):
        t = t_v[pl.ds(cch * 16, 16)]
        inc = plsc.cumsum(t)
        excl = inc - t + jnp.full((16,), carry, jnp.int32)
        counters[pl.ds(cch * 16, 16)] = excl + a_v[pl.ds(cch * 16, 16)]
        return carry + _vsum(t)
    lax.fori_loop(0, _NB // 16, prefix, jnp.int32(0))

    cbase0 = s * (_TP // 16)
    spbase = c * _HALF

    def window(win, _):
        base = cbase0 + win * _PWIN
        pltpu.sync_copy(keys_in.at[pl.ds(base, _PWIN)], win_k)
        pltpu.sync_copy(vals_in.at[pl.ds(base, _PWIN)], win_x)
        for j in range(_PWIN // 128):
            for jj in range(8):
                i = j * 8 + jj
                k = win_k[pl.ds(i * 16, 16)]
                dig = lax.shift_right_logical(k, jnp.int32(shift)) & jnp.int32(_NB - 1)
                occ, last = plsc.scan_count(dig)
                b = plsc.load_gather(counters, [dig])
                dest = b + occ + jnp.int32(-_OCC_BASE)
                plsc.store_scatter(counters, [dig],
                                   b + occ + jnp.int32(1 - _OCC_BASE), mask=last)
                spd = dest - jnp.full((16,), spbase, jnp.int32)
                valid = (spd >= 0) & (spd < jnp.int32(_HALF))
                dump = jnp.full((16,), _HALF, jnp.int32) + (
                    lanes + jnp.int32((i & 15) * 16))
                spd = jnp.where(valid, spd, dump)
                dest_v[j, pl.ds(jj * 16, 16)] = spd
        cps = []
        for j in range(_PWIN // 128):
            cps.append(pltpu.async_copy(
                win_k.at[pl.ds(j * 128, 128)],
                keys_sp.at[dest_v.at[j]], sem))
            cps.append(pltpu.async_copy(
                win_x.at[pl.ds(j * 128, 128)],
                vals_sp.at[dest_v.at[j]], sem))
        for cp in cps:
            cp.wait()
        return 0

    lax.fori_loop(0, _TP // 16 // _PWIN, window, 0)
    plsc.subcore_barrier()

    off = s * _EXPW
    pltpu.sync_copy(keys_sp.at[pl.ds(off, _EXPW)],
                    keys_out.at[pl.ds(spbase + off, _EXPW)])
    pltpu.sync_copy(vals_sp.at[pl.ds(off, _EXPW)],
                    vals_out.at[pl.ds(spbase + off, _EXPW)])


def _run_permute(shift, keys_in, vals_in, hist_p):
    kern = pl.kernel(
        functools.partial(_permute_kernel, shift),
        compiler_params=_params(),
        out_type=(
            jax.ShapeDtypeStruct((_TP,), jnp.int32),
            jax.ShapeDtypeStruct((_TP,), jnp.float32),
        ),
        mesh=_mesh(),
        scratch_types=[
            pltpu.VMEM((_NW, 256), jnp.int32),
            pltpu.VMEM((_NB,), jnp.int32),
            pltpu.VMEM((_NB,), jnp.int32),
            pltpu.VMEM((_NB,), jnp.int32),
            pltpu.VMEM((_PWIN,), jnp.int32),
            pltpu.VMEM((_PWIN,), jnp.float32),
            pltpu.VMEM((_PWIN // 128, 128), jnp.int32),
            pltpu.VMEM_SHARED((_HALF + 512,), jnp.int32),
            pltpu.VMEM_SHARED((_HALF + 512,), jnp.float32),
            pltpu.SemaphoreType.DMA,
        ],
    )
    return kern(keys_in, vals_in, hist_p)


def _coalesce_kernel(n_total, keys_h, xvals_h, vals_h, lo_h, hi_h,
                     win_k, win_x, dest_v, cnt_v, vbuf, lbuf, zero_v, zero_i,
                     vacc_sp, lacc_sp, cnt_sp, sem):
    c = lax.axis_index("c")
    s = lax.axis_index("s")
    lanes = _lanes()
    zeros16 = jnp.zeros((16,), jnp.int32)

    @pl.when(c == 0)
    def _():
        def zv(i, _):
            zero_v[pl.ds(i * 16, 16)] = jnp.zeros((16,), jnp.float32)
            zero_i[pl.ds(i * 16, 16)] = jnp.zeros((16,), jnp.int32)
            return 0
        lax.fori_loop(0, _OWIN // 16, zv, 0)

        def zwin(i, _):
            off = s * (_SPN // 16) + i * _OWIN
            pltpu.sync_copy(zero_v, vacc_sp.at[pl.ds(off, _OWIN)])
            pltpu.sync_copy(zero_i, lacc_sp.at[pl.ds(off, _OWIN)])
            return 0
        lax.fori_loop(0, _SPN // 16 // _OWIN, zwin, 0)

        cbase0 = s * (_TP // 16)

        @pl.when(s > 0)
        def _():
            pltpu.sync_copy(keys_h.at[pl.ds(cbase0 - 8, 8)],
                            win_k.at[pl.ds(0, 8)])
        prevv = win_k[pl.ds(0, 16)]
        prev0 = jnp.where(
            (jnp.full((16,), 1, jnp.int32) * (s > 0).astype(jnp.int32)) > 0,
            jnp.full((16,), _vsum(jnp.where(lanes == 7, prevv, zeros16)),
                     jnp.int32),
            jnp.full((16,), -2, jnp.int32))

        shiftperm = jnp.maximum(lanes - 1, 0)
        gdn = lax.GatherDimensionNumbers(
            offset_dims=(), collapsed_slice_dims=(0,), start_index_map=(0,))

        def boundaries(k, prev_carry, base_i):
            kshift = lax.gather(k, shiftperm.reshape(16, 1), gdn, (1,),
                                mode=lax.GatherScatterMode.PROMISE_IN_BOUNDS)
            prev = jnp.where(lanes == 0, jnp.full((16,), prev_carry, jnp.int32),
                             kshift)
            gpos = jnp.full((16,), base_i, jnp.int32) + lanes
            isn = ((k != prev) & (gpos > 0) & (gpos < jnp.int32(_TOTAL)))
            new_carry = _vsum(jnp.where(lanes == 15, k, zeros16))
            return isn.astype(jnp.int32), new_carry

        p0 = _vsum(jnp.where(lanes == 0, prev0, zeros16))

        def awin(win, carry):
            nb, pk = carry
            base = cbase0 + win * _PWIN
            pltpu.sync_copy(keys_h.at[pl.ds(base, _PWIN)], win_k)

            def avreg(i, carry2):
                nb2, pk2 = carry2
                k = win_k[pl.ds(i * 16, 16)]
                isn, pk3 = boundaries(k, pk2, base + i * 16)
                return nb2 + _vsum(isn), pk3
            return lax.fori_loop(0, _PWIN // 16, avreg, (nb, pk))

        nbound, _pk = lax.fori_loop(0, _TP // 16 // _PWIN, awin,
                                    (jnp.int32(0), p0))

        cnt_v[0, pl.ds(0, 16)] = jnp.full((16,), nbound, jnp.int32)
        pltpu.sync_copy(cnt_v.at[0], cnt_sp.at[s])
        plsc.subcore_barrier()
        pltpu.sync_copy(cnt_sp, cnt_v)

        offs = jnp.int32(0)
        nseg = jnp.int32(0)
        for ws in range(16):
            v = cnt_v[ws, pl.ds(0, 16)]
            cw = _vsum(jnp.where(lanes == 0, v, zeros16))
            offs = offs + cw * (jnp.int32(ws) < s).astype(jnp.int32)
            nseg = nseg + cw
        nseg = nseg + 1

        def bwin(win, carry):
            seg0, pk = carry
            base = cbase0 + win * _PWIN
            pltpu.sync_copy(keys_h.at[pl.ds(base, _PWIN)], win_k)
            pltpu.sync_copy(xvals_h.at[pl.ds(base, _PWIN)], win_x)

            def bvreg(i, carry2):
                seg1, pk2 = carry2
                k = win_k[pl.ds(i * 16, 16)]
                v = win_x[pl.ds(i * 16, 16)]
                isn, pk3 = boundaries(k, pk2, base + i * 16)
                segv = jnp.full((16,), seg1, jnp.int32) + plsc.cumsum(isn)
                gpos = jnp.full((16,), base + i * 16, jnp.int32) + lanes
                valid = gpos < jnp.int32(_TOTAL)
                sv = lax.shift_right_logical(k, jnp.int32(16))
                dv = k & jnp.int32(0xFFFF)
                lo = sv * jnp.int32(n_total) + dv
                dump = jnp.full((16,), _DUMP, jnp.int32) + (gpos & jnp.int32(255))
                dest = jnp.where(valid, segv, dump)
                vbuf[pl.ds(i * 16, 16)] = jnp.where(
                    valid, v, jnp.zeros((16,), jnp.float32))
                lbuf[pl.ds(i * 16, 16)] = lo
                jrow = lax.shift_right_logical(i, 3)
                dest_v[jrow, pl.ds((i & 7) * 16, 16)] = dest
                return seg1 + _vsum(isn), pk3

            seg2, pk2 = lax.fori_loop(0, _PWIN // 16, bvreg, (seg0, pk))
            cps = []
            for j in range(_PWIN // 128):
                cps.append(pltpu.async_copy(
                    vbuf.at[pl.ds(j * 128, 128)],
                    vacc_sp.at[dest_v.at[j]], sem, add=True))
                cps.append(pltpu.async_copy(
                    lbuf.at[pl.ds(j * 128, 128)],
                    lacc_sp.at[dest_v.at[j]], sem))
            for cp in cps:
                cp.wait()
            return seg2, pk2

        lax.fori_loop(0, _TP // 16 // _PWIN, bwin, (offs, p0))
        plsc.subcore_barrier()

        def cwin(i, _):
            off = s * _OC + i * _OWIN
            pltpu.sync_copy(vacc_sp.at[pl.ds(off, _OWIN)],
                            vbuf.at[pl.ds(0, _OWIN)])
            pltpu.sync_copy(vbuf.at[pl.ds(0, _OWIN)],
                            vals_h.at[pl.ds(off, _OWIN)])
            pltpu.sync_copy(lacc_sp.at[pl.ds(off, _OWIN)],
                            lbuf.at[pl.ds(0, _OWIN)])
            pltpu.sync_copy(lbuf.at[pl.ds(0, _OWIN)],
                            lo_h.at[pl.ds(off, _OWIN)])

            def hveq(q, _):
                gpos = jnp.full((16,), off + q * 16, jnp.int32) + lanes
                hi = jnp.where(gpos < nseg, zeros16,
                               jnp.full((16,), _MIN32, jnp.int32))
                lbuf[pl.ds(q * 16, 16)] = hi
                return 0
            lax.fori_loop(0, _OWIN // 16, hveq, 0)
            pltpu.sync_copy(lbuf.at[pl.ds(0, _OWIN)], hi_h.at[pl.ds(off, _OWIN)])
            return 0
        lax.fori_loop(0, _OC // _OWIN, cwin, 0)


def _run_coalesce(n_total, keys, xvals):
    kern = pl.kernel(
        functools.partial(_coalesce_kernel, n_total),
        compiler_params=_params(),
        out_type=(
            jax.ShapeDtypeStruct((_OUTP,), jnp.float32),
            jax.ShapeDtypeStruct((_OUTP,), jnp.int32),
            jax.ShapeDtypeStruct((_OUTP,), jnp.int32),
        ),
        mesh=_mesh(),
        scratch_types=[
            pltpu.VMEM((_PWIN,), jnp.int32),
            pltpu.VMEM((_PWIN,), jnp.float32),
            pltpu.VMEM((16, 128), jnp.int32),
            pltpu.VMEM((16, 16), jnp.int32),
            pltpu.VMEM((_PWIN,), jnp.float32),
            pltpu.VMEM((_PWIN,), jnp.int32),
            pltpu.VMEM((_OWIN,), jnp.float32),
            pltpu.VMEM((_OWIN,), jnp.int32),
            pltpu.VMEM_SHARED((_SPN,), jnp.float32),
            pltpu.VMEM_SHARED((_SPN,), jnp.int32),
            pltpu.VMEM_SHARED((16, 16), jnp.int32),
            pltpu.SemaphoreType.DMA,
        ],
    )
    return kern(keys, xvals)


def kernel(all_emb, edge_index, graph_vals, W1, b1, W2, b2, rnd_idx, rnd_vals):
    n, d = all_emb.shape
    e = edge_index.shape[1]
    r = rnd_idx.shape[1]
    total = e + r

    with jax.enable_x64(False):
        gi, l2 = _noise_constants(e)
        a_tab, b_tab = _precompute_ab(all_emb, W1, b1)

        src_p = jnp.pad(edge_index[0].astype(jnp.int32), (0, _EP - e))
        dst_p = jnp.pad(edge_index[1].astype(jnp.int32), (0, _EP - e))
        gi_p = jnp.pad(gi, (0, _EP - e))
        l2_p = jnp.pad(l2, (0, _EP - e))
        ru = jnp.pad(rnd_idx[0].astype(jnp.int32), (0, _RP - r))
        ri = jnp.pad(rnd_idx[1].astype(jnp.int32), (0, _RP - r))
        rv = jnp.pad(rnd_vals.astype(jnp.float32), (0, _RP - r))
        w2f = W2[:, 0].astype(jnp.float32)
        b2v = jnp.full((16,), b2[0], jnp.float32)

        keys, xvals = _run_edges(a_tab, b_tab, src_p, dst_p, gi_p, l2_p,
                                 w2f, b2v, ru, ri, rv)
        for sh in _SHIFTS:
            hist = _run_hist(sh, keys)
            keys, xvals = _run_permute(sh, keys, xvals, hist)

        vals_p, lo_p, hi_p = _run_coalesce(n, keys, xvals)

    out_vals = vals_p[:total]
    lin = (hi_p[:total].astype(jnp.int64) << 32) | (
        lo_p[:total].astype(jnp.uint32).astype(jnp.int64))
    return out_vals, lin
